# baseline (speedup 1.0000x reference)
import jax
import jax.numpy as jnp
from jax import lax
from jax.experimental import pallas as pl
from jax.experimental.pallas import tpu as pltpu
from jax.experimental.pallas import tpu_sc as plsc

N = 65536
M = 512
L = 16
NC = 2
NS = 16
NW = NC * NS
MB = M * L

NSC = 16384
NTC = N - NSC
RPW = NSC // NW
GB = 4
NB = RPW // (L * GB)
UNR = 8

CH = 32
RT = NTC // 128


def _sqrt16(x):
    i = lax.bitcast_convert_type(x, jnp.int32)
    i = jnp.int32(0x1FBD1DF5) + lax.shift_right_logical(i, 1)
    y = lax.bitcast_convert_type(i, jnp.float32)
    half = jnp.float32(0.5)
    y = half * (y + x / y)
    y = half * (y + x / y)
    y = half * (y + x / y)
    return y


def _sc_body(colors_h, palb_h, out_h, vx, vy, vz, wx, wy, wz, ws, tot, sem):
    c = lax.axis_index("c")
    s = lax.axis_index("s")
    wid = s * NC + c
    base = wid * RPW
    copies = [
        pltpu.make_async_copy(colors_h.at[pl.ds(base, RPW)], vx, sem),
        pltpu.make_async_copy(colors_h.at[pl.ds(N + base, RPW)], vy, sem),
        pltpu.make_async_copy(colors_h.at[pl.ds(2 * N + base, RPW)], vz, sem),
        pltpu.make_async_copy(palb_h.at[pl.ds(0, MB)], wx, sem),
        pltpu.make_async_copy(palb_h.at[pl.ds(MB, MB)], wy, sem),
        pltpu.make_async_copy(palb_h.at[pl.ds(2 * MB, MB)], wz, sem),
        pltpu.make_async_copy(palb_h.at[pl.ds(3 * MB, MB)], ws, sem),
    ]
    for cp in copies:
        cp.start()
    for cp in copies:
        cp.wait()

    total = jnp.zeros((L,), jnp.float32)
    for b in range(NB):
        xs, ys, zs = [], [], []
        for g in range(GB):
            off = (b * GB + g) * L
            xs.append(vx[pl.ds(off, L)])
            ys.append(vy[pl.ds(off, L)])
            zs.append(vz[pl.ds(off, L)])
        init = tuple(jnp.full((L,), jnp.inf, jnp.float32) for _ in range(GB))

        def inner(i, bests, xs=xs, ys=ys, zs=zs):
            for k in range(UNR):
                o = pl.multiple_of((i * UNR + k) * L, L)
                px = wx[pl.ds(o, L)]
                py = wy[pl.ds(o, L)]
                pz = wz[pl.ds(o, L)]
                sv = ws[pl.ds(o, L)]
                new = []
                for g in range(GB):
                    v = (xs[g] * px + ys[g] * py) + (zs[g] * pz + sv)
                    new.append(jnp.minimum(bests[g], v))
                bests = tuple(new)
            return bests

        bests = lax.fori_loop(0, M // UNR, inner, init)
        for g in range(GB):
            r2 = xs[g] * xs[g] + ys[g] * ys[g] + zs[g] * zs[g]
            d2 = jnp.maximum(bests[g] + r2, jnp.float32(0.0))
            total = total + _sqrt16(d2)
    tot[...] = total
    pltpu.sync_copy(tot, out_h.at[wid])


def _tc_body(col_ref, palb_ref, o_ref):
    acc = jnp.zeros((CH, 128), jnp.float32)
    for ci in range(RT // CH):
        off = NSC // 128 + ci * CH
        X = col_ref[0, pl.ds(off, CH), :]
        Y = col_ref[1, pl.ds(off, CH), :]
        Z = col_ref[2, pl.ds(off, CH), :]

        def ent(i, best, X=X, Y=Y, Z=Z):
            for k in range(32):
                j = i * 32 + k
                px = palb_ref[0, j]
                py = palb_ref[1, j]
                pz = palb_ref[2, j]
                sv = palb_ref[3, j]
                v = (X * px + Y * py) + (Z * pz + sv)
                best = jnp.minimum(best, v)
            return best

        best = lax.fori_loop(0, M // 32, ent, jnp.full((CH, 128), jnp.inf, jnp.float32))
        r2 = X * X + Y * Y + Z * Z
        acc = acc + jnp.sqrt(jnp.maximum(best + r2, jnp.float32(0.0)))
    o_ref[0, 0] = jnp.sum(acc)


def _combine_body(p_ref, t_ref, o_ref):
    o_ref[0, 0] = (jnp.sum(p_ref[...]) + t_ref[0, 0]) * jnp.float32(1.0 / N)


@jax.jit
def kernel(output_colors, target_palette):
    pal_t = target_palette.T
    s_row = jnp.sum(pal_t * pal_t, axis=0, keepdims=True)
    palq4 = jnp.concatenate([jnp.float32(-2.0) * pal_t, s_row], axis=0)
    colors_flat = output_colors.T.reshape(-1)
    palq_b = jnp.repeat(palq4, L, axis=1).reshape(-1)
    palb = jnp.broadcast_to(palq4[:, :, None], (4, M, 128))
    mesh = plsc.VectorSubcoreMesh(
        core_axis_name="c", subcore_axis_name="s",
        num_cores=NC, num_subcores=NS)
    sc = pl.kernel(
        _sc_body,
        out_type=jax.ShapeDtypeStruct((NW, L), jnp.float32),
        mesh=mesh,
        scratch_types=[
            pltpu.VMEM((RPW,), jnp.float32),
            pltpu.VMEM((RPW,), jnp.float32),
            pltpu.VMEM((RPW,), jnp.float32),
            pltpu.VMEM((MB,), jnp.float32),
            pltpu.VMEM((MB,), jnp.float32),
            pltpu.VMEM((MB,), jnp.float32),
            pltpu.VMEM((MB,), jnp.float32),
            pltpu.VMEM((L,), jnp.float32),
            pltpu.SemaphoreType.DMA,
        ],
    )
    partials = sc(colors_flat, palq_b)

    colors_3d = colors_flat.reshape(3, N // 128, 128)
    tc_sum = pl.pallas_call(
        _tc_body,
        out_shape=jax.ShapeDtypeStruct((1, 1), jnp.float32),
        in_specs=[
            pl.BlockSpec(memory_space=pltpu.VMEM),
            pl.BlockSpec(memory_space=pltpu.VMEM),
        ],
        out_specs=pl.BlockSpec(memory_space=pltpu.SMEM),
    )(colors_3d, palb)

    out = pl.pallas_call(
        _combine_body,
        out_shape=jax.ShapeDtypeStruct((1, 1), jnp.float32),
        in_specs=[
            pl.BlockSpec(memory_space=pltpu.VMEM),
            pl.BlockSpec(memory_space=pltpu.SMEM),
        ],
        out_specs=pl.BlockSpec(memory_space=pltpu.SMEM),
    )(partials, tc_sum)
    return out[0, 0]


# speedup vs baseline: 2.3281x; 2.3281x over previous
import jax
import jax.numpy as jnp
from jax import lax
from jax.experimental import pallas as pl
from jax.experimental.pallas import tpu as pltpu
from jax.experimental.pallas import tpu_sc as plsc

N = 65536
M = 512
L = 16
NC = 2
NS = 16
NW = NC * NS
MB = M * L

NSC = 16384
NTC = N - NSC
RPW = NSC // NW
GB = 4
NB = RPW // (L * GB)
UNR = 4

CH = 32
RT = NTC // 128


def _sqrt16(x):
    i = lax.bitcast_convert_type(x, jnp.int32)
    i = jnp.int32(0x1FBD1DF5) + lax.shift_right_logical(i, 1)
    y = lax.bitcast_convert_type(i, jnp.float32)
    half = jnp.float32(0.5)
    y = half * (y + x / y)
    y = half * (y + x / y)
    y = half * (y + x / y)
    return y


def _sc_body(colors_h, palb_h, out_h, vx, vy, vz, wx, wy, wz, ws, tot, sem):
    c = lax.axis_index("c")
    s = lax.axis_index("s")
    wid = s * NC + c
    base = wid * RPW
    copies = [
        pltpu.make_async_copy(colors_h.at[pl.ds(base, RPW)], vx, sem),
        pltpu.make_async_copy(colors_h.at[pl.ds(N + base, RPW)], vy, sem),
        pltpu.make_async_copy(colors_h.at[pl.ds(2 * N + base, RPW)], vz, sem),
        pltpu.make_async_copy(palb_h.at[pl.ds(0, MB)], wx, sem),
        pltpu.make_async_copy(palb_h.at[pl.ds(MB, MB)], wy, sem),
        pltpu.make_async_copy(palb_h.at[pl.ds(2 * MB, MB)], wz, sem),
        pltpu.make_async_copy(palb_h.at[pl.ds(3 * MB, MB)], ws, sem),
    ]
    for cp in copies:
        cp.start()
    for cp in copies:
        cp.wait()

    total = jnp.zeros((L,), jnp.float32)
    for b in range(NB):
        xs, ys, zs = [], [], []
        for g in range(GB):
            off = (b * GB + g) * L
            xs.append(vx[pl.ds(off, L)])
            ys.append(vy[pl.ds(off, L)])
            zs.append(vz[pl.ds(off, L)])
        init = tuple(jnp.full((L,), jnp.inf, jnp.float32) for _ in range(GB))

        def inner(i, bests, xs=xs, ys=ys, zs=zs):
            for k in range(UNR):
                o = pl.multiple_of((i * UNR + k) * L, L)
                px = wx[pl.ds(o, L)]
                py = wy[pl.ds(o, L)]
                pz = wz[pl.ds(o, L)]
                sv = ws[pl.ds(o, L)]
                new = []
                for g in range(GB):
                    v = (xs[g] * px + ys[g] * py) + (zs[g] * pz + sv)
                    new.append(jnp.minimum(bests[g], v))
                bests = tuple(new)
            return bests

        bests = lax.fori_loop(0, M // UNR, inner, init)
        for g in range(GB):
            r2 = xs[g] * xs[g] + ys[g] * ys[g] + zs[g] * zs[g]
            d2 = jnp.maximum(bests[g] + r2, jnp.float32(0.0))
            total = total + _sqrt16(d2)
    tot[...] = total
    pltpu.sync_copy(tot, out_h.at[wid])


def _tc_body(col_ref, palb_ref, o_ref):
    acc = jnp.zeros((CH, 128), jnp.float32)
    for ci in range(RT // CH):
        off = NSC // 128 + ci * CH
        X = col_ref[0, pl.ds(off, CH), :]
        Y = col_ref[1, pl.ds(off, CH), :]
        Z = col_ref[2, pl.ds(off, CH), :]

        def ent(i, best, X=X, Y=Y, Z=Z):
            for k in range(32):
                j = i * 32 + k
                px = palb_ref[0, j]
                py = palb_ref[1, j]
                pz = palb_ref[2, j]
                sv = palb_ref[3, j]
                v = (X * px + Y * py) + (Z * pz + sv)
                best = jnp.minimum(best, v)
            return best

        best = lax.fori_loop(0, M // 32, ent, jnp.full((CH, 128), jnp.inf, jnp.float32))
        r2 = X * X + Y * Y + Z * Z
        acc = acc + jnp.sqrt(jnp.maximum(best + r2, jnp.float32(0.0)))
    o_ref[0, 0] = jnp.sum(acc)


def _combine_body(p_ref, t_ref, o_ref):
    o_ref[0, 0] = (jnp.sum(p_ref[...]) + t_ref[0, 0]) * jnp.float32(1.0 / N)


@jax.jit
def kernel(output_colors, target_palette):
    pal_t = target_palette.T
    s_row = jnp.sum(pal_t * pal_t, axis=0, keepdims=True)
    palq4 = jnp.concatenate([jnp.float32(-2.0) * pal_t, s_row], axis=0)
    colors_flat = output_colors.T.reshape(-1)
    palq_b = jnp.repeat(palq4, L, axis=1).reshape(-1)
    palb = jnp.broadcast_to(palq4[:, :, None], (4, M, 128))
    mesh = plsc.VectorSubcoreMesh(
        core_axis_name="c", subcore_axis_name="s",
        num_cores=NC, num_subcores=NS)
    sc = pl.kernel(
        _sc_body,
        out_type=jax.ShapeDtypeStruct((NW, L), jnp.float32),
        mesh=mesh,
        scratch_types=[
            pltpu.VMEM((RPW,), jnp.float32),
            pltpu.VMEM((RPW,), jnp.float32),
            pltpu.VMEM((RPW,), jnp.float32),
            pltpu.VMEM((MB,), jnp.float32),
            pltpu.VMEM((MB,), jnp.float32),
            pltpu.VMEM((MB,), jnp.float32),
            pltpu.VMEM((MB,), jnp.float32),
            pltpu.VMEM((L,), jnp.float32),
            pltpu.SemaphoreType.DMA,
        ],
    )
    partials = sc(colors_flat, palq_b)

    colors_3d = colors_flat.reshape(3, N // 128, 128)
    tc_sum = pl.pallas_call(
        _tc_body,
        out_shape=jax.ShapeDtypeStruct((1, 1), jnp.float32),
        in_specs=[
            pl.BlockSpec(memory_space=pltpu.VMEM),
            pl.BlockSpec(memory_space=pltpu.VMEM),
        ],
        out_specs=pl.BlockSpec(memory_space=pltpu.SMEM),
    )(colors_3d, palb)

    out = pl.pallas_call(
        _combine_body,
        out_shape=jax.ShapeDtypeStruct((1, 1), jnp.float32),
        in_specs=[
            pl.BlockSpec(memory_space=pltpu.VMEM),
            pl.BlockSpec(memory_space=pltpu.SMEM),
        ],
        out_specs=pl.BlockSpec(memory_space=pltpu.SMEM),
    )(partials, tc_sum)
    return out[0, 0]


# speedup vs baseline: 2.4603x; 1.0568x over previous
import jax
import jax.numpy as jnp
from jax import lax
from jax.experimental import pallas as pl
from jax.experimental.pallas import tpu as pltpu
from jax.experimental.pallas import tpu_sc as plsc

N = 65536
M = 512
L = 16
NC = 2
NS = 16
NW = NC * NS
MB = M * L

NSC = 14336
NTC = N - NSC
RPW = NSC // NW
GB = 4
NB = RPW // (L * GB)
UNR = 4

CH = 32
RT = NTC // 128


def _sqrt16(x):
    i = lax.bitcast_convert_type(x, jnp.int32)
    i = jnp.int32(0x1FBD1DF5) + lax.shift_right_logical(i, 1)
    y = lax.bitcast_convert_type(i, jnp.float32)
    half = jnp.float32(0.5)
    y = half * (y + x / y)
    y = half * (y + x / y)
    y = half * (y + x / y)
    return y


def _sc_body(colors_h, palb_h, out_h, vx, vy, vz, wx, wy, wz, ws, tot, sem):
    c = lax.axis_index("c")
    s = lax.axis_index("s")
    wid = s * NC + c
    base = wid * RPW
    copies = [
        pltpu.make_async_copy(colors_h.at[pl.ds(base, RPW)], vx, sem),
        pltpu.make_async_copy(colors_h.at[pl.ds(N + base, RPW)], vy, sem),
        pltpu.make_async_copy(colors_h.at[pl.ds(2 * N + base, RPW)], vz, sem),
        pltpu.make_async_copy(palb_h.at[pl.ds(0, MB)], wx, sem),
        pltpu.make_async_copy(palb_h.at[pl.ds(MB, MB)], wy, sem),
        pltpu.make_async_copy(palb_h.at[pl.ds(2 * MB, MB)], wz, sem),
        pltpu.make_async_copy(palb_h.at[pl.ds(3 * MB, MB)], ws, sem),
    ]
    for cp in copies:
        cp.start()
    for cp in copies:
        cp.wait()

    total = jnp.zeros((L,), jnp.float32)
    for b in range(NB):
        xs, ys, zs = [], [], []
        for g in range(GB):
            off = (b * GB + g) * L
            xs.append(vx[pl.ds(off, L)])
            ys.append(vy[pl.ds(off, L)])
            zs.append(vz[pl.ds(off, L)])
        init = tuple(jnp.full((L,), jnp.inf, jnp.float32) for _ in range(GB))

        def inner(i, bests, xs=xs, ys=ys, zs=zs):
            for k in range(UNR):
                o = pl.multiple_of((i * UNR + k) * L, L)
                px = wx[pl.ds(o, L)]
                py = wy[pl.ds(o, L)]
                pz = wz[pl.ds(o, L)]
                sv = ws[pl.ds(o, L)]
                new = []
                for g in range(GB):
                    v = (xs[g] * px + ys[g] * py) + (zs[g] * pz + sv)
                    new.append(jnp.minimum(bests[g], v))
                bests = tuple(new)
            return bests

        bests = lax.fori_loop(0, M // UNR, inner, init)
        for g in range(GB):
            r2 = xs[g] * xs[g] + ys[g] * ys[g] + zs[g] * zs[g]
            d2 = jnp.maximum(bests[g] + r2, jnp.float32(0.0))
            total = total + _sqrt16(d2)
    tot[...] = total
    pltpu.sync_copy(tot, out_h.at[wid])


def _tc_body(col_ref, palb_ref, o_ref):
    acc = jnp.zeros((128,), jnp.float32)
    starts = list(range(0, RT - CH + 1, CH))
    rem = RT - (len(starts) * CH)
    chunks = [(s, CH) for s in starts] + ([(len(starts) * CH, rem)] if rem else [])
    for (cs, cw) in chunks:
        off = NSC // 128 + cs
        X = col_ref[0, pl.ds(off, cw), :]
        Y = col_ref[1, pl.ds(off, cw), :]
        Z = col_ref[2, pl.ds(off, cw), :]

        def ent(i, best, X=X, Y=Y, Z=Z):
            for k in range(32):
                j = i * 32 + k
                px = palb_ref[0, j]
                py = palb_ref[1, j]
                pz = palb_ref[2, j]
                sv = palb_ref[3, j]
                v = (X * px + Y * py) + (Z * pz + sv)
                best = jnp.minimum(best, v)
            return best

        best = lax.fori_loop(0, M // 32, ent, jnp.full((cw, 128), jnp.inf, jnp.float32))
        r2 = X * X + Y * Y + Z * Z
        d = jnp.sqrt(jnp.maximum(best + r2, jnp.float32(0.0)))
        acc = acc + jnp.sum(d, axis=0)
    o_ref[0, 0] = jnp.sum(acc)


def _combine_body(p_ref, t_ref, o_ref):
    o_ref[0, 0] = (jnp.sum(p_ref[...]) + t_ref[0, 0]) * jnp.float32(1.0 / N)


@jax.jit
def kernel(output_colors, target_palette):
    pal_t = target_palette.T
    s_row = jnp.sum(pal_t * pal_t, axis=0, keepdims=True)
    palq4 = jnp.concatenate([jnp.float32(-2.0) * pal_t, s_row], axis=0)
    colors_flat = output_colors.T.reshape(-1)
    palq_b = jnp.repeat(palq4, L, axis=1).reshape(-1)
    palb = jnp.broadcast_to(palq4[:, :, None], (4, M, 128))
    mesh = plsc.VectorSubcoreMesh(
        core_axis_name="c", subcore_axis_name="s",
        num_cores=NC, num_subcores=NS)
    sc = pl.kernel(
        _sc_body,
        out_type=jax.ShapeDtypeStruct((NW, L), jnp.float32),
        mesh=mesh,
        scratch_types=[
            pltpu.VMEM((RPW,), jnp.float32),
            pltpu.VMEM((RPW,), jnp.float32),
            pltpu.VMEM((RPW,), jnp.float32),
            pltpu.VMEM((MB,), jnp.float32),
            pltpu.VMEM((MB,), jnp.float32),
            pltpu.VMEM((MB,), jnp.float32),
            pltpu.VMEM((MB,), jnp.float32),
            pltpu.VMEM((L,), jnp.float32),
            pltpu.SemaphoreType.DMA,
        ],
    )
    partials = sc(colors_flat, palq_b)

    colors_3d = colors_flat.reshape(3, N // 128, 128)
    tc_sum = pl.pallas_call(
        _tc_body,
        out_shape=jax.ShapeDtypeStruct((1, 1), jnp.float32),
        in_specs=[
            pl.BlockSpec(memory_space=pltpu.VMEM),
            pl.BlockSpec(memory_space=pltpu.VMEM),
        ],
        out_specs=pl.BlockSpec(memory_space=pltpu.SMEM),
    )(colors_3d, palb)

    out = pl.pallas_call(
        _combine_body,
        out_shape=jax.ShapeDtypeStruct((1, 1), jnp.float32),
        in_specs=[
            pl.BlockSpec(memory_space=pltpu.VMEM),
            pl.BlockSpec(memory_space=pltpu.SMEM),
        ],
        out_specs=pl.BlockSpec(memory_space=pltpu.SMEM),
    )(partials, tc_sum)
    return out[0, 0]


# speedup vs baseline: 2.4666x; 1.0026x over previous
import jax
import jax.numpy as jnp
from jax import lax
from jax.experimental import pallas as pl
from jax.experimental.pallas import tpu as pltpu
from jax.experimental.pallas import tpu_sc as plsc

N = 65536
M = 512
L = 16
NC = 2
NS = 16
NW = NC * NS
MB = M * L

NSC = 14336
NTC = N - NSC
RPW = NSC // NW
GB = 4
NB = RPW // (L * GB)
UNR = 4

CH = 32
RT = NTC // 128


def _sqrt16(x):
    i = lax.bitcast_convert_type(x, jnp.int32)
    i = jnp.int32(0x1FBD1DF5) + lax.shift_right_logical(i, 1)
    y = lax.bitcast_convert_type(i, jnp.float32)
    half = jnp.float32(0.5)
    y = half * (y + x / y)
    y = half * (y + x / y)
    y = half * (y + x / y)
    return y


def _sc_body(colors_h, palb_h, out_h, vx, vy, vz, wx, wy, wz, ws, tot, sem):
    c = lax.axis_index("c")
    s = lax.axis_index("s")
    wid = s * NC + c
    base = wid * RPW
    copies = [
        pltpu.make_async_copy(colors_h.at[pl.ds(base, RPW)], vx, sem),
        pltpu.make_async_copy(colors_h.at[pl.ds(N + base, RPW)], vy, sem),
        pltpu.make_async_copy(colors_h.at[pl.ds(2 * N + base, RPW)], vz, sem),
        pltpu.make_async_copy(palb_h.at[pl.ds(0, MB)], wx, sem),
        pltpu.make_async_copy(palb_h.at[pl.ds(MB, MB)], wy, sem),
        pltpu.make_async_copy(palb_h.at[pl.ds(2 * MB, MB)], wz, sem),
        pltpu.make_async_copy(palb_h.at[pl.ds(3 * MB, MB)], ws, sem),
    ]
    for cp in copies:
        cp.start()
    for cp in copies:
        cp.wait()

    total = jnp.zeros((L,), jnp.float32)
    for b in range(NB):
        xs, ys, zs = [], [], []
        for g in range(GB):
            off = (b * GB + g) * L
            xs.append(vx[pl.ds(off, L)])
            ys.append(vy[pl.ds(off, L)])
            zs.append(vz[pl.ds(off, L)])
        init = tuple(jnp.full((L,), jnp.inf, jnp.float32) for _ in range(GB))

        def inner(i, bests, xs=xs, ys=ys, zs=zs):
            for k in range(UNR):
                o = pl.multiple_of((i * UNR + k) * L, L)
                px = wx[pl.ds(o, L)]
                py = wy[pl.ds(o, L)]
                pz = wz[pl.ds(o, L)]
                sv = ws[pl.ds(o, L)]
                new = []
                for g in range(GB):
                    v = (xs[g] * px + ys[g] * py) + (zs[g] * pz + sv)
                    new.append(jnp.minimum(bests[g], v))
                bests = tuple(new)
            return bests

        bests = lax.fori_loop(0, M // UNR, inner, init)
        for g in range(GB):
            r2 = xs[g] * xs[g] + ys[g] * ys[g] + zs[g] * zs[g]
            d2 = jnp.maximum(bests[g] + r2, jnp.float32(0.0))
            total = total + _sqrt16(d2)
    tot[...] = total
    pltpu.sync_copy(tot, out_h.at[wid])


def _tc_body(col_ref, palb_ref, o_ref):
    acc = jnp.zeros((128,), jnp.float32)
    starts = list(range(0, RT - CH + 1, CH))
    rem = RT - (len(starts) * CH)
    chunks = [(s, CH) for s in starts] + ([(len(starts) * CH, rem)] if rem else [])
    for (cs, cw) in chunks:
        off = NSC // 128 + cs
        X = col_ref[0, pl.ds(off, cw), :]
        Y = col_ref[1, pl.ds(off, cw), :]
        Z = col_ref[2, pl.ds(off, cw), :]

        def ent(i, best, X=X, Y=Y, Z=Z):
            for k in range(64):
                j = i * 64 + k
                px = palb_ref[0, j]
                py = palb_ref[1, j]
                pz = palb_ref[2, j]
                sv = palb_ref[3, j]
                v = (X * px + Y * py) + (Z * pz + sv)
                best = jnp.minimum(best, v)
            return best

        best = lax.fori_loop(0, M // 64, ent, jnp.full((cw, 128), jnp.inf, jnp.float32))
        r2 = X * X + Y * Y + Z * Z
        d = jnp.sqrt(jnp.maximum(best + r2, jnp.float32(0.0)))
        acc = acc + jnp.sum(d, axis=0)
    o_ref[0, 0] = jnp.sum(acc)


def _combine_body(p_ref, t_ref, o_ref):
    o_ref[0, 0] = (jnp.sum(p_ref[...]) + t_ref[0, 0]) * jnp.float32(1.0 / N)


@jax.jit
def kernel(output_colors, target_palette):
    pal_t = target_palette.T
    s_row = jnp.sum(pal_t * pal_t, axis=0, keepdims=True)
    palq4 = jnp.concatenate([jnp.float32(-2.0) * pal_t, s_row], axis=0)
    colors_flat = output_colors.T.reshape(-1)
    palq_b = jnp.repeat(palq4, L, axis=1).reshape(-1)
    palb = jnp.broadcast_to(palq4[:, :, None], (4, M, 128))
    mesh = plsc.VectorSubcoreMesh(
        core_axis_name="c", subcore_axis_name="s",
        num_cores=NC, num_subcores=NS)
    sc = pl.kernel(
        _sc_body,
        out_type=jax.ShapeDtypeStruct((NW, L), jnp.float32),
        mesh=mesh,
        scratch_types=[
            pltpu.VMEM((RPW,), jnp.float32),
            pltpu.VMEM((RPW,), jnp.float32),
            pltpu.VMEM((RPW,), jnp.float32),
            pltpu.VMEM((MB,), jnp.float32),
            pltpu.VMEM((MB,), jnp.float32),
            pltpu.VMEM((MB,), jnp.float32),
            pltpu.VMEM((MB,), jnp.float32),
            pltpu.VMEM((L,), jnp.float32),
            pltpu.SemaphoreType.DMA,
        ],
    )
    partials = sc(colors_flat, palq_b)

    colors_3d = colors_flat.reshape(3, N // 128, 128)
    tc_sum = pl.pallas_call(
        _tc_body,
        out_shape=jax.ShapeDtypeStruct((1, 1), jnp.float32),
        in_specs=[
            pl.BlockSpec(memory_space=pltpu.VMEM),
            pl.BlockSpec(memory_space=pltpu.VMEM),
        ],
        out_specs=pl.BlockSpec(memory_space=pltpu.SMEM),
    )(colors_3d, palb)

    out = pl.pallas_call(
        _combine_body,
        out_shape=jax.ShapeDtypeStruct((1, 1), jnp.float32),
        in_specs=[
            pl.BlockSpec(memory_space=pltpu.VMEM),
            pl.BlockSpec(memory_space=pltpu.SMEM),
        ],
        out_specs=pl.BlockSpec(memory_space=pltpu.SMEM),
    )(partials, tc_sum)
    return out[0, 0]
